# baseline (speedup 1.0000x reference)
import functools

import jax
import jax.numpy as jnp
from jax.experimental import pallas as pl

B, N, T, S = 1024, 148, 64, 32
DIM = 64

BB = 32


def _elu(x):
    return jnp.where(x > 0, x, jnp.exp(x) - 1.0)


def _fused_kernel(phi_ref, y_ref, g_ref, W1_ref, b1_ref, W2_ref, b2_ref,
                  C1_ref, cb1_ref, C2_ref, cb2_ref, C3_ref, cb3_ref, out_ref):
    phi = phi_ref[...]
    yv = y_ref[...]
    g = g_ref[...]

    mn_phi = jnp.min(jnp.where(phi == 0.0, 100.0, phi), axis=2)
    mx_phi = jnp.max(phi, axis=2)
    mn_y = jnp.min(jnp.where(yv == 0.0, 100.0, yv), axis=1)
    mx_y = jnp.max(yv, axis=1)
    mn = jnp.minimum(mn_phi, mn_y)
    mx = jnp.maximum(mx_phi, mx_y)

    eye = (jax.lax.broadcasted_iota(jnp.int32, (N, N), 0) ==
           jax.lax.broadcasted_iota(jnp.int32, (N, N), 1)).astype(jnp.float32)
    A = g + eye[None, :, :]
    ones_col = jnp.ones((N, 1), jnp.float32)
    deg = jax.lax.dot_general(A, ones_col, (((2,), (0,)), ((), ())),
                              preferred_element_type=jnp.float32)
    dinv = jax.lax.rsqrt(deg[:, :, 0])

    mn_s = mn * dinv
    mx_s = mx * dinv
    p_mn = jnp.sum(A * mn_s[:, None, :], axis=2) * dinv
    p_mx = jnp.sum(A * mx_s[:, None, :], axis=2) * dinv
    W1r0 = W1_ref[0:1, :]
    W1r1 = W1_ref[1:2, :]
    h1 = _elu(p_mn[:, :, None] * W1r0[None] + p_mx[:, :, None] * W1r1[None]
              + b1_ref[...][None])

    hs = h1 * dinv[:, :, None]
    u = jax.lax.dot_general(A, hs, (((2,), (1,)), ((0,), (0,))),
                            preferred_element_type=jnp.float32)
    x2 = u * dinv[:, :, None]
    t2 = jax.lax.dot_general(x2, W2_ref[...], (((2,), (0,)), ((), ())),
                             preferred_element_type=jnp.float32)
    h2 = _elu(t2 + b2_ref[...][None])
    pooled = jnp.sum(h2, axis=1) * (1.0 / N)

    z = _elu(jnp.dot(pooled, C1_ref[...], preferred_element_type=jnp.float32)
             + cb1_ref[...])
    z = _elu(jnp.dot(z, C2_ref[...], preferred_element_type=jnp.float32)
             + cb2_ref[...])
    out_ref[...] = (jnp.dot(z, C3_ref[...], preferred_element_type=jnp.float32)
                    + cb3_ref[...])


@functools.partial(jax.jit, static_argnums=())
def kernel(phi, y, g, W1, b1, W2, b2, C1, cb1, C2, cb2, C3, cb3):
    b1r = b1.reshape(1, -1)
    b2r = b2.reshape(1, -1)
    cb1r = cb1.reshape(1, -1)
    cb2r = cb2.reshape(1, -1)
    cb3r = cb3.reshape(1, -1)

    grid = (B // BB,)
    wspec = lambda shape: pl.BlockSpec(shape, lambda i: (0,) * len(shape))
    out = pl.pallas_call(
        _fused_kernel,
        grid=grid,
        in_specs=[
            pl.BlockSpec((BB, N, T), lambda i: (i, 0, 0)),
            pl.BlockSpec((BB, S, N), lambda i: (i, 0, 0)),
            pl.BlockSpec((BB, N, N), lambda i: (i, 0, 0)),
            wspec(W1.shape),
            wspec(b1r.shape),
            wspec(W2.shape),
            wspec(b2r.shape),
            wspec(C1.shape),
            wspec(cb1r.shape),
            wspec(C2.shape),
            wspec(cb2r.shape),
            wspec(C3.shape),
            wspec(cb3r.shape),
        ],
        out_specs=pl.BlockSpec((BB, 2), lambda i: (i, 0)),
        out_shape=jax.ShapeDtypeStruct((B, 2), jnp.float32),
    )(phi, y, g, W1, b1r, W2, b2r, C1, cb1r, C2, cb2r, C3, cb3r)
    return out


# speedup vs baseline: 1.0087x; 1.0087x over previous
import functools

import jax
import jax.numpy as jnp
from jax.experimental import pallas as pl

B, N, T, S = 1024, 148, 64, 32
DIM = 64

BB = 32


def _elu(x):
    return jnp.where(x > 0, x, jnp.exp(x) - 1.0)


def _fused_kernel(phi_ref, y_ref, g_ref, W1_ref, b1_ref, W2_ref, b2_ref,
                  C1_ref, cb1_ref, C2_ref, cb2_ref, C3_ref, cb3_ref, out_ref):
    phi = phi_ref[...]
    yv = y_ref[...]
    g = g_ref[...]

    mn_phi = jnp.min(jnp.where(phi == 0.0, 100.0, phi), axis=2,
                     keepdims=True)
    mx_phi = jnp.max(phi, axis=2, keepdims=True)
    mn_y = jnp.min(jnp.where(yv == 0.0, 100.0, yv), axis=1,
                   keepdims=True)
    mx_y = jnp.max(yv, axis=1, keepdims=True)
    mn3 = jnp.minimum(mn_phi, jnp.transpose(mn_y, (0, 2, 1)))
    mx3 = jnp.maximum(mx_phi, jnp.transpose(mx_y, (0, 2, 1)))

    ones_col = jnp.ones((N, 1), jnp.float32)
    deg3 = jax.lax.dot_general(g, ones_col, (((2,), (0,)), ((), ())),
                               preferred_element_type=jnp.float32) + 1.0
    dinv3 = jax.lax.rsqrt(deg3)

    r2 = jnp.concatenate([mn3, mx3], axis=2) * dinv3
    u1 = jax.lax.dot_general(g, r2, (((2,), (1,)), ((0,), (0,))),
                             preferred_element_type=jnp.float32) + r2
    p_mn = u1[:, :, 0:1] * dinv3
    p_mx = u1[:, :, 1:2] * dinv3
    W1r0 = W1_ref[0:1, :]
    W1r1 = W1_ref[1:2, :]
    h1 = _elu(p_mn * W1r0[None] + p_mx * W1r1[None]
              + b1_ref[...][None])

    hs = h1 * dinv3
    u = jax.lax.dot_general(g, hs, (((2,), (1,)), ((0,), (0,))),
                            preferred_element_type=jnp.float32) + hs
    x2 = u * dinv3
    t2 = jax.lax.dot_general(x2, W2_ref[...], (((2,), (0,)), ((), ())),
                             preferred_element_type=jnp.float32)
    h2 = _elu(t2 + b2_ref[...][None])
    pooled = jnp.sum(h2, axis=1) * (1.0 / N)

    z = _elu(jnp.dot(pooled, C1_ref[...], preferred_element_type=jnp.float32)
             + cb1_ref[...])
    z = _elu(jnp.dot(z, C2_ref[...], preferred_element_type=jnp.float32)
             + cb2_ref[...])
    out_ref[...] = (jnp.dot(z, C3_ref[...], preferred_element_type=jnp.float32)
                    + cb3_ref[...])


@functools.partial(jax.jit, static_argnums=())
def kernel(phi, y, g, W1, b1, W2, b2, C1, cb1, C2, cb2, C3, cb3):
    b1r = b1.reshape(1, -1)
    b2r = b2.reshape(1, -1)
    cb1r = cb1.reshape(1, -1)
    cb2r = cb2.reshape(1, -1)
    cb3r = cb3.reshape(1, -1)

    grid = (B // BB,)
    wspec = lambda shape: pl.BlockSpec(shape, lambda i: (0,) * len(shape))
    out = pl.pallas_call(
        _fused_kernel,
        grid=grid,
        in_specs=[
            pl.BlockSpec((BB, N, T), lambda i: (i, 0, 0)),
            pl.BlockSpec((BB, S, N), lambda i: (i, 0, 0)),
            pl.BlockSpec((BB, N, N), lambda i: (i, 0, 0)),
            wspec(W1.shape),
            wspec(b1r.shape),
            wspec(W2.shape),
            wspec(b2r.shape),
            wspec(C1.shape),
            wspec(cb1r.shape),
            wspec(C2.shape),
            wspec(cb2r.shape),
            wspec(C3.shape),
            wspec(cb3r.shape),
        ],
        out_specs=pl.BlockSpec((BB, 2), lambda i: (i, 0)),
        out_shape=jax.ShapeDtypeStruct((B, 2), jnp.float32),
    )(phi, y, g, W1, b1r, W2, b2r, C1, cb1r, C2, cb2r, C3, cb3r)
    return out
